# baseline (speedup 1.0000x reference)
import jax
import jax.numpy as jnp
from jax import lax
from jax.experimental import pallas as pl
from jax.experimental.pallas import tpu as pltpu

_NEG_TO_POS = 3
_DET_W = 0.5
_VAL_W = 0.5
_WX, _WY, _WW, _WH = 10.0, 10.0, 5.0, 5.0
_SENT = -0x80000000


def _phase_a(logits_ref, mi_sub_ref, labels_ref, tab_ref, mi_lane_ref,
             anc_ref, reg_ref, cls_out_ref, bbox_out_ref):
    n = pl.program_id(1)
    T, C = logits_ref.shape[2], logits_ref.shape[3]
    G = labels_ref.shape[2]

    logits = logits_ref[0, 0]
    mi_s = mi_sub_ref[0, 0]
    fg_s = mi_s >= 0
    safe_s = jnp.clip(mi_s, 0, G - 1)
    ohg = (safe_s == lax.broadcasted_iota(jnp.int32, (T, G), 1))
    label_f = jnp.sum(jnp.where(ohg, labels_ref[0], 0.0), axis=1,
                      keepdims=True)
    tgt = jnp.where(fg_s, label_f.astype(jnp.int32), 0)
    mx = jnp.max(logits, axis=1, keepdims=True)
    ssum = jnp.sum(jnp.exp(logits - mx), axis=1, keepdims=True)
    lse = mx + jnp.log(ssum)
    ohc = (tgt == lax.broadcasted_iota(jnp.int32, (T, C), 1))
    t_logit = jnp.sum(jnp.where(ohc, logits, 0.0), axis=1, keepdims=True)
    cls_out_ref[0, 0] = (lse - t_logit).reshape(1, T)

    mi_l = mi_lane_ref[0, 0]
    fg_l = (mi_l >= 0).astype(jnp.float32)
    safe_l = jnp.clip(mi_l, 0, G - 1)
    ohT = (safe_l == lax.broadcasted_iota(jnp.int32, (G, T), 0)
           ).astype(jnp.bfloat16)
    gath = lax.dot_general(tab_ref[0], ohT, (((1,), (0,)), ((), ())),
                           preferred_element_type=jnp.float32)
    gt_cx = gath[0:1] + gath[1:2] + gath[2:3]
    gt_cy = gath[3:4] + gath[4:5] + gath[5:6]
    gt_lw = gath[6:7] + gath[7:8] + gath[8:9]
    gt_lh = gath[9:10] + gath[10:11] + gath[11:12]

    anc = anc_ref[0, 0]
    x1, y1, x2, y2 = anc[0:1], anc[1:2], anc[2:3], anc[3:4]
    ex_w = x2 - x1
    ex_h = y2 - y1
    ex_cx = x1 + 0.5 * ex_w
    ex_cy = y1 + 0.5 * ex_h
    dx = _WX * (gt_cx - ex_cx) / ex_w
    dy = _WY * (gt_cy - ex_cy) / ex_h
    dw = _WW * (gt_lw - jnp.log(ex_w))
    dh = _WH * (gt_lh - jnp.log(ex_h))
    tgt4 = jnp.concatenate([dx, dy, dw, dh], axis=0)
    d = jnp.abs(reg_ref[0, 0] - tgt4)
    sl1 = jnp.where(d < 1.0, 0.5 * d * d, d - 0.5)
    chunk = jnp.sum(sl1 * fg_l)

    @pl.when(n == 0)
    def _():
        bbox_out_ref[...] = jnp.zeros_like(bbox_out_ref)
    bbox_out_ref[...] += chunk


def _phase_b(cls_ref, mi_ref, bbox_ref, rej_ref, lab_ref, out_ref):
    B, N = cls_ref.shape
    v = cls_ref[...]
    mi = mi_ref[...]
    fg = mi >= 0
    num_fg_row = jnp.sum(fg.astype(jnp.int32), axis=1, keepdims=True)
    num_fg_total = jnp.sum(num_fg_row.astype(jnp.float32))
    nf = jnp.maximum(1.0, num_fg_total)
    sum_fg = jnp.sum(jnp.where(fg, v, 0.0))

    bits = lax.bitcast_convert_type(v, jnp.int32)
    key = jnp.where(bits >= 0, bits, bits ^ jnp.int32(0x7FFFFFFF))
    key = jnp.where(fg, jnp.int32(_SENT), key)

    kk = jnp.minimum(_NEG_TO_POS * num_fg_row, N)
    kk1 = jnp.maximum(kk, 1)

    lo0 = jnp.full((B, 1), _SENT, jnp.int32)
    hi0 = jnp.full((B, 1), 0x7FFFFFFF, jnp.int32)

    def vbody(_, lohi):
        lo, hi = lohi
        mid = (lo >> 1) + (hi >> 1) + (lo & hi & 1)
        cnt = jnp.sum((key >= mid).astype(jnp.int32), axis=1, keepdims=True)
        pred = cnt >= kk1
        return jnp.where(pred, mid, lo), jnp.where(pred, hi, mid)

    t, _ = lax.fori_loop(0, 32, vbody, (lo0, hi0))
    gt_mask = key > t
    c_gt = jnp.sum(gt_mask.astype(jnp.int32), axis=1, keepdims=True)
    sum_gt = jnp.sum(jnp.where(gt_mask, v, 0.0), axis=1, keepdims=True)
    m = jnp.maximum(kk - c_gt, 0)
    tb = jnp.where(t >= 0, t, t ^ jnp.int32(0x7FFFFFFF))
    tval = lax.bitcast_convert_type(tb, jnp.float32)

    iot = lax.broadcasted_iota(jnp.int32, (B, N), 1)

    def ibody(_, lohi):
        lo, hi = lohi
        mid = (lo + hi) >> 1
        cnt = jnp.sum((fg & (iot < mid)).astype(jnp.int32), axis=1,
                      keepdims=True)
        pred = cnt >= m
        return jnp.where(pred, lo, mid + 1), jnp.where(pred, mid, hi)

    _, jstar = lax.fori_loop(0, 15, ibody,
                             (jnp.zeros((B, 1), jnp.int32),
                              jnp.full((B, 1), N, jnp.int32)))
    extra = jnp.sum(jnp.where(fg & (iot < jstar), v, 0.0), axis=1,
                    keepdims=True)

    bg_row = sum_gt + jnp.where(t == jnp.int32(_SENT), extra,
                                m.astype(jnp.float32) * tval)
    sum_bg = jnp.sum(jnp.where(kk > 0, bg_row, 0.0))

    bbox_total = jnp.sum(bbox_ref[:, 0:1, 0:1])
    regression = bbox_total / nf
    classification = (sum_fg + sum_bg) / nf

    rl = rej_ref[...]
    l0, l1 = rl[:, 0:1], rl[:, 1:2]
    mx2 = jnp.maximum(l0, l1)
    lse2 = mx2 + jnp.log(jnp.exp(l0 - mx2) + jnp.exp(l1 - mx2))
    tsel = jnp.where(lab_ref[...] == 0, l0, l1)
    val = jnp.mean(lse2 - tsel)

    loss = _DET_W * (regression + classification) + _VAL_W * val
    lane = lax.broadcasted_iota(jnp.int32, (8, 128), 1)
    out_ref[...] = jnp.where(
        lane == 0, loss,
        jnp.where(lane == 1, regression,
                  jnp.where(lane == 2, classification,
                            jnp.where(lane == 3, val, 0.0))))


def _split3(x):
    h = x.astype(jnp.bfloat16)
    r = x - h.astype(jnp.float32)
    m = r.astype(jnp.bfloat16)
    l = (r - m.astype(jnp.float32)).astype(jnp.bfloat16)
    return h, m, l


def kernel(boxes, labels, image_label, bbox_regression, cls_logits, anchors,
           rejection_logits, matched_idxs):
    B, N, C = cls_logits.shape
    G = boxes.shape[1]
    NB = 8
    T = N // NB
    f32 = jnp.float32
    mi = matched_idxs.astype(jnp.int32)

    x1, y1 = boxes[..., 0], boxes[..., 1]
    gw = boxes[..., 2] - x1
    gh = boxes[..., 3] - y1
    rows = []
    for a in (x1 + 0.5 * gw, y1 + 0.5 * gh, jnp.log(gw), jnp.log(gh)):
        rows += list(_split3(a))
    rows += [jnp.zeros_like(rows[0])] * 4
    tab = jnp.stack(rows, axis=1)

    labels_f = labels.astype(f32).reshape(B, 1, G)
    mi_sub = mi.reshape(B, NB, T, 1)
    mi_lane = mi.reshape(B, NB, 1, T)
    anc_l = anchors.transpose(0, 2, 1).reshape(B, 4, NB, T).transpose(0, 2, 1, 3)
    reg_l = bbox_regression.transpose(0, 2, 1).reshape(B, 4, NB, T).transpose(0, 2, 1, 3)

    cls_loss_r, bbox_part = pl.pallas_call(
        _phase_a,
        grid=(B, NB),
        in_specs=[
            pl.BlockSpec((1, 1, T, C), lambda b, n: (b, n, 0, 0)),
            pl.BlockSpec((1, 1, T, 1), lambda b, n: (b, n, 0, 0)),
            pl.BlockSpec((1, 1, G), lambda b, n: (b, 0, 0)),
            pl.BlockSpec((1, 16, G), lambda b, n: (b, 0, 0)),
            pl.BlockSpec((1, 1, 1, T), lambda b, n: (b, n, 0, 0)),
            pl.BlockSpec((1, 1, 4, T), lambda b, n: (b, n, 0, 0)),
            pl.BlockSpec((1, 1, 4, T), lambda b, n: (b, n, 0, 0)),
        ],
        out_specs=[
            pl.BlockSpec((1, 1, 1, T), lambda b, n: (b, n, 0, 0)),
            pl.BlockSpec((1, 8, 128), lambda b, n: (b, 0, 0)),
        ],
        out_shape=[
            jax.ShapeDtypeStruct((B, NB, 1, T), f32),
            jax.ShapeDtypeStruct((B, 8, 128), f32),
        ],
    )(cls_logits.reshape(B, NB, T, C), mi_sub, labels_f, tab, mi_lane, anc_l,
      reg_l)

    out = pl.pallas_call(
        _phase_b,
        in_specs=[
            pl.BlockSpec((B, N), lambda: (0, 0)),
            pl.BlockSpec((B, N), lambda: (0, 0)),
            pl.BlockSpec((B, 8, 128), lambda: (0, 0, 0)),
            pl.BlockSpec((B, 2), lambda: (0, 0)),
            pl.BlockSpec((B, 1), lambda: (0, 0)),
        ],
        out_specs=pl.BlockSpec((8, 128), lambda: (0, 0)),
        out_shape=jax.ShapeDtypeStruct((8, 128), f32),
    )(cls_loss_r.reshape(B, N), mi, bbox_part, rejection_logits,
      image_label.astype(jnp.int32).reshape(B, 1))

    return (out[0, 0], out[0, 1], out[0, 2], out[0, 3])


# speedup vs baseline: 1.1365x; 1.1365x over previous
import jax
import jax.numpy as jnp
from jax import lax
from jax.experimental import pallas as pl
from jax.experimental.pallas import tpu as pltpu

_NEG_TO_POS = 3
_DET_W = 0.5
_VAL_W = 0.5
_WX, _WY, _WW, _WH = 10.0, 10.0, 5.0, 5.0
_SENT = -0x80000000


def _taa(x, idx, axis):
    other = tuple(d for d in range(2) if d != axis)
    dnums = lax.GatherDimensionNumbers(
        offset_dims=(), collapsed_slice_dims=(axis,), start_index_map=(axis,),
        operand_batching_dims=other, start_indices_batching_dims=other)
    return lax.gather(x, idx[..., None], dnums, slice_sizes=(1, 1),
                      mode=lax.GatherScatterMode.PROMISE_IN_BOUNDS)


def _phase_a(logits_ref, mi_sub_ref, labels_ref, tab_ref, mi_lane_ref,
             anc_ref, reg_ref, cls_out_ref, bbox_out_ref):
    n = pl.program_id(1)
    T, C = logits_ref.shape[2], logits_ref.shape[3]
    G = labels_ref.shape[2]

    logits = logits_ref[0, 0]
    mi_s = mi_sub_ref[0, 0]
    fg_s = mi_s >= 0
    safe_s = jnp.clip(mi_s, 0, G - 1)
    labels_b = jnp.broadcast_to(labels_ref[0], (T, G))
    label_f = _taa(labels_b, safe_s, 1)
    tgt = jnp.where(fg_s, label_f.astype(jnp.int32), 0)
    mblk = jnp.max(logits)
    e = jnp.exp(logits - mblk)
    ssum = lax.dot_general(e.astype(jnp.bfloat16),
                           jnp.ones((C, 1), jnp.bfloat16),
                           (((1,), (0,)), ((), ())),
                           preferred_element_type=jnp.float32)
    lse = mblk + jnp.log(ssum)
    t_logit = _taa(logits, tgt, 1)
    cls_out_ref[0] = lse - t_logit

    mi_l = mi_lane_ref[0, 0]
    fg_l = (mi_l >= 0).astype(jnp.float32)
    safe_l = jnp.clip(mi_l, 0, G - 1)
    tab = tab_ref[0]
    gt_cx = _taa(tab[0:1], safe_l, 1)
    gt_cy = _taa(tab[1:2], safe_l, 1)
    gt_lw = _taa(tab[2:3], safe_l, 1)
    gt_lh = _taa(tab[3:4], safe_l, 1)

    anc = anc_ref[0, 0]
    x1, y1, x2, y2 = anc[0:1], anc[1:2], anc[2:3], anc[3:4]
    ex_w = x2 - x1
    ex_h = y2 - y1
    ex_cx = x1 + 0.5 * ex_w
    ex_cy = y1 + 0.5 * ex_h
    dx = _WX * (gt_cx - ex_cx) / ex_w
    dy = _WY * (gt_cy - ex_cy) / ex_h
    dw = _WW * (gt_lw - jnp.log(ex_w))
    dh = _WH * (gt_lh - jnp.log(ex_h))
    tgt4 = jnp.concatenate([dx, dy, dw, dh], axis=0)
    d = jnp.abs(reg_ref[0, 0] - tgt4)
    sl1 = jnp.where(d < 1.0, 0.5 * d * d, d - 0.5)
    chunk = jnp.sum(sl1 * fg_l)

    @pl.when(n == 0)
    def _():
        bbox_out_ref[...] = jnp.zeros_like(bbox_out_ref)
    bbox_out_ref[...] += chunk


def _phase_b(cls_ref, mi_ref, bbox_ref, rej_ref, lab_ref, out_ref):
    B, N = cls_ref.shape
    v = cls_ref[...]
    mi = mi_ref[...]
    fg = mi >= 0
    num_fg_row = jnp.sum(fg.astype(jnp.int32), axis=1, keepdims=True)
    num_fg_total = jnp.sum(num_fg_row.astype(jnp.float32))
    nf = jnp.maximum(1.0, num_fg_total)
    sum_fg = jnp.sum(jnp.where(fg, v, 0.0))

    bits = lax.bitcast_convert_type(v, jnp.int32)
    key = jnp.where(bits >= 0, bits, bits ^ jnp.int32(0x7FFFFFFF))
    key = jnp.where(fg, jnp.int32(_SENT), key)

    kk = jnp.minimum(_NEG_TO_POS * num_fg_row, N)
    kk1 = jnp.maximum(kk, 1)

    lo0 = jnp.full((B, 1), _SENT, jnp.int32)
    hi0 = jnp.full((B, 1), 0x7FFFFFFF, jnp.int32)

    def vbody(_, lohi):
        lo, hi = lohi
        mid = (lo >> 1) + (hi >> 1) + (lo & hi & 1)
        cnt = jnp.sum((key >= mid).astype(jnp.int32), axis=1, keepdims=True)
        pred = cnt >= kk1
        return jnp.where(pred, mid, lo), jnp.where(pred, hi, mid)

    t, _ = lax.fori_loop(0, 32, vbody, (lo0, hi0))
    gt_mask = key > t
    c_gt = jnp.sum(gt_mask.astype(jnp.int32), axis=1, keepdims=True)
    sum_gt = jnp.sum(jnp.where(gt_mask, v, 0.0), axis=1, keepdims=True)
    m = jnp.maximum(kk - c_gt, 0)
    tb = jnp.where(t >= 0, t, t ^ jnp.int32(0x7FFFFFFF))
    tval = lax.bitcast_convert_type(tb, jnp.float32)

    iot = lax.broadcasted_iota(jnp.int32, (B, N), 1)

    def ibody(_, lohi):
        lo, hi = lohi
        mid = (lo + hi) >> 1
        cnt = jnp.sum((fg & (iot < mid)).astype(jnp.int32), axis=1,
                      keepdims=True)
        pred = cnt >= m
        return jnp.where(pred, lo, mid + 1), jnp.where(pred, mid, hi)

    _, jstar = lax.fori_loop(0, 15, ibody,
                             (jnp.zeros((B, 1), jnp.int32),
                              jnp.full((B, 1), N, jnp.int32)))
    extra = jnp.sum(jnp.where(fg & (iot < jstar), v, 0.0), axis=1,
                    keepdims=True)

    bg_row = sum_gt + jnp.where(t == jnp.int32(_SENT), extra,
                                m.astype(jnp.float32) * tval)
    sum_bg = jnp.sum(jnp.where(kk > 0, bg_row, 0.0))

    bbox_total = jnp.sum(bbox_ref[:, 0:1, 0:1])
    regression = bbox_total / nf
    classification = (sum_fg + sum_bg) / nf

    rl = rej_ref[...]
    l0, l1 = rl[:, 0:1], rl[:, 1:2]
    mx2 = jnp.maximum(l0, l1)
    lse2 = mx2 + jnp.log(jnp.exp(l0 - mx2) + jnp.exp(l1 - mx2))
    tsel = jnp.where(lab_ref[...] == 0, l0, l1)
    val = jnp.mean(lse2 - tsel)

    loss = _DET_W * (regression + classification) + _VAL_W * val
    lane = lax.broadcasted_iota(jnp.int32, (8, 128), 1)
    out_ref[...] = jnp.where(
        lane == 0, loss,
        jnp.where(lane == 1, regression,
                  jnp.where(lane == 2, classification,
                            jnp.where(lane == 3, val, 0.0))))


def kernel(boxes, labels, image_label, bbox_regression, cls_logits, anchors,
           rejection_logits, matched_idxs):
    B, N, C = cls_logits.shape
    G = boxes.shape[1]
    NB = 4
    T = N // NB
    f32 = jnp.float32
    mi = matched_idxs.astype(jnp.int32)

    x1, y1 = boxes[..., 0], boxes[..., 1]
    gw = boxes[..., 2] - x1
    gh = boxes[..., 3] - y1
    tab = jnp.stack([x1 + 0.5 * gw, y1 + 0.5 * gh, jnp.log(gw), jnp.log(gh)],
                    axis=1)

    labels_f = labels.astype(f32).reshape(B, 1, G)
    mi_sub = mi.reshape(B, NB, T, 1)
    mi_lane = mi.reshape(B, NB, 1, T)
    anc_l = anchors.transpose(0, 2, 1).reshape(B, 4, NB, T).transpose(0, 2, 1, 3)
    reg_l = bbox_regression.transpose(0, 2, 1).reshape(B, 4, NB, T).transpose(0, 2, 1, 3)

    cls_loss_r, bbox_part = pl.pallas_call(
        _phase_a,
        grid=(B, NB),
        in_specs=[
            pl.BlockSpec((1, 1, T, C), lambda b, n: (b, n, 0, 0)),
            pl.BlockSpec((1, 1, T, 1), lambda b, n: (b, n, 0, 0)),
            pl.BlockSpec((1, 1, G), lambda b, n: (b, 0, 0)),
            pl.BlockSpec((1, 4, G), lambda b, n: (b, 0, 0)),
            pl.BlockSpec((1, 1, 1, T), lambda b, n: (b, n, 0, 0)),
            pl.BlockSpec((1, 1, 4, T), lambda b, n: (b, n, 0, 0)),
            pl.BlockSpec((1, 1, 4, T), lambda b, n: (b, n, 0, 0)),
        ],
        out_specs=[
            pl.BlockSpec((1, T, 1), lambda b, n: (b * NB + n, 0, 0)),
            pl.BlockSpec((1, 8, 128), lambda b, n: (b, 0, 0)),
        ],
        out_shape=[
            jax.ShapeDtypeStruct((B * NB, T, 1), f32),
            jax.ShapeDtypeStruct((B, 8, 128), f32),
        ],
    )(cls_logits.reshape(B, NB, T, C), mi_sub, labels_f, tab, mi_lane, anc_l,
      reg_l)

    out = pl.pallas_call(
        _phase_b,
        in_specs=[
            pl.BlockSpec((B, N), lambda: (0, 0)),
            pl.BlockSpec((B, N), lambda: (0, 0)),
            pl.BlockSpec((B, 8, 128), lambda: (0, 0, 0)),
            pl.BlockSpec((B, 2), lambda: (0, 0)),
            pl.BlockSpec((B, 1), lambda: (0, 0)),
        ],
        out_specs=pl.BlockSpec((8, 128), lambda: (0, 0)),
        out_shape=jax.ShapeDtypeStruct((8, 128), f32),
    )(cls_loss_r.reshape(B, N), mi, bbox_part, rejection_logits,
      image_label.astype(jnp.int32).reshape(B, 1))

    return (out[0, 0], out[0, 1], out[0, 2], out[0, 3])
